# baseline (speedup 1.0000x reference)
import math

import jax
import jax.numpy as jnp
from jax import lax
from jax.experimental import pallas as pl
from jax.experimental.pallas import tpu as pltpu
from jax.experimental.pallas import tpu_sc as plsc

NC, NS, L = 2, 16, 16
NW = NC * NS

IDXW = 128
KPC = 2
CHUNK = KPC * IDXW
NBUF = 4


def _sc_gather_body(tokens_hbm, table_hbm, s_hbm, idx_bufs, row_bufs,
                    gsems, osems):
    D = table_hbm.shape[1]
    HIST, n_idx_rows, _ = tokens_hbm.shape
    BATCH = n_idx_rows * IDXW
    per_w = BATCH // NW
    cph = per_w // CHUNK
    n_chunks = HIST * cph
    n_outer = n_chunks // NBUF
    wid = lax.axis_index("s") * NC + lax.axis_index("c")

    def start_gather(c, b):
        h = c // cph
        part = c % cph
        idx_r0 = wid * (per_w // IDXW) + part * KPC
        pltpu.sync_copy(tokens_hbm.at[h, pl.ds(idx_r0, KPC)], idx_bufs[b])
        for k in range(KPC):
            pltpu.async_copy(table_hbm.at[idx_bufs[b].at[k]],
                             row_bufs[b].at[pl.ds(k * IDXW, IDXW)], gsems[b])

    def wait_gather(b):
        pltpu.make_async_copy(table_hbm.at[pl.ds(0, CHUNK)], row_bufs[b],
                              gsems[b]).wait()

    def start_out(c, b):
        h = c // cph
        part = c % cph
        row0 = h * BATCH + wid * per_w + part * CHUNK
        pltpu.async_copy(row_bufs[b], s_hbm.at[pl.ds(row0, CHUNK)], osems[b])

    def wait_out(b):
        pltpu.make_async_copy(row_bufs[b], s_hbm.at[pl.ds(0, CHUNK)],
                              osems[b]).wait()

    def step(c, k, do_gather, do_outwait):
        if do_gather:
            bg = (k + 2) % NBUF
            if do_outwait:
                wait_out(bg)
            start_gather(c + 2, bg)
        wait_gather(k)
        start_out(c, k)

    start_gather(0, 0)
    start_gather(1, 1)
    step(0, 0, True, False)
    step(1, 1, True, False)
    step(2, 2, True, True)
    step(3, 3, True, True)

    def outer(i, carry):
        c0 = i * NBUF
        for k in range(NBUF):
            step(c0 + k, k, True, True)
        return carry

    lax.fori_loop(1, n_outer - 1, outer, 0)

    c0 = (n_outer - 1) * NBUF
    step(c0 + 0, 0, True, True)
    step(c0 + 1, 1, True, True)
    step(c0 + 2, 2, False, False)
    step(c0 + 3, 3, False, False)

    for b in range(NBUF):
        wait_out(b)


def _tc_transpose_body(s_ref, o_ref):
    D = o_ref.shape[1]
    half = o_ref.shape[2] // 2
    n, _, w = s_ref.shape
    x2 = s_ref[...].reshape(n * 8, w)
    scale = math.sqrt(D)
    o_ref[0, :, :half] = jnp.transpose(x2[:, :D]) * scale
    o_ref[0, :, half:] = jnp.transpose(x2[:, D:]) * scale


def _tc_transpose_body_alias(s_ref, al_ref, o_ref):
    del al_ref
    _tc_transpose_body(s_ref, o_ref)


NPHASE = 4


def kernel(tokens, table):
    BATCH, HIST = tokens.shape
    V, D = table.shape
    hp = HIST // NPHASE
    per_w = BATCH // NW
    assert BATCH % (NW * CHUNK) == 0 and D % L == 0
    assert HIST % NPHASE == 0 and (hp * per_w) // CHUNK >= 3 * NBUF

    run_gather = pl.kernel(
        _sc_gather_body,
        out_type=jax.ShapeDtypeStruct((hp * BATCH, D), jnp.float32),
        mesh=plsc.VectorSubcoreMesh(core_axis_name="c", subcore_axis_name="s"),
        scratch_types=[
            [pltpu.VMEM((KPC, IDXW), jnp.int32) for _ in range(NBUF)],
            [pltpu.VMEM((CHUNK, D), jnp.float32) for _ in range(NBUF)],
            [pltpu.SemaphoreType.DMA for _ in range(NBUF)],
            [pltpu.SemaphoreType.DMA for _ in range(NBUF)],
        ],
        compiler_params=pltpu.CompilerParams(use_tc_tiling_on_sc=False),
    )
    tokens_il = (jnp.transpose(tokens)
                 .reshape(HIST, 2, BATCH // 2)
                 .transpose(0, 2, 1)
                 .reshape(HIST, BATCH // IDXW, IDXW))
    tokens_il = lax.optimization_barrier(tokens_il)

    rows_per_slab = BATCH * D // (8 * 128)
    out_t = None
    for p in range(NPHASE):
        tok_p = lax.slice_in_dim(tokens_il, p * hp, (p + 1) * hp, axis=0)
        s = run_gather(tok_p, table)
        s5 = s.reshape(hp * rows_per_slab, 8, 128)
        in_specs = [pl.BlockSpec((rows_per_slab, 8, 128),
                                 lambda h: (h, 0, 0))]
        operands = [s5]
        kwargs = {}
        body = _tc_transpose_body
        if p > 0:
            in_specs.append(pl.BlockSpec(memory_space=pl.ANY))
            operands.append(out_t)
            kwargs["input_output_aliases"] = {1: 0}
            body = _tc_transpose_body_alias
        out_t = pl.pallas_call(
            body,
            grid=(hp,),
            in_specs=in_specs,
            out_specs=pl.BlockSpec((1, D, BATCH),
                                   lambda h, p=p: (p * hp + h, 0, 0)),
            out_shape=jax.ShapeDtypeStruct((HIST, D, BATCH), jnp.float32),
            **kwargs,
        )(*operands)
    return jnp.transpose(out_t, (2, 0, 1))


# speedup vs baseline: 1.0107x; 1.0107x over previous
import math

import jax
import jax.numpy as jnp
from jax import lax
from jax.experimental import pallas as pl
from jax.experimental.pallas import tpu as pltpu
from jax.experimental.pallas import tpu_sc as plsc

NC, NS, L = 2, 16, 16
NW = NC * NS

IDXW = 128
KPC = 2
CHUNK = KPC * IDXW
NBUF = 4


def _sc_gather_body(tokens_hbm, table_hbm, s_hbm, idx_bufs, row_bufs,
                    gsems, osems):
    D = table_hbm.shape[1]
    HIST, n_idx_rows, _ = tokens_hbm.shape
    BATCH = n_idx_rows * IDXW
    per_w = BATCH // NW
    cph = per_w // CHUNK
    n_chunks = HIST * cph
    n_outer = n_chunks // NBUF
    wid = lax.axis_index("s") * NC + lax.axis_index("c")

    def start_gather(c, b):
        h = c // cph
        part = c % cph
        idx_r0 = wid * (per_w // IDXW) + part * KPC
        pltpu.sync_copy(tokens_hbm.at[h, pl.ds(idx_r0, KPC)], idx_bufs[b])
        for k in range(KPC):
            pltpu.async_copy(table_hbm.at[idx_bufs[b].at[k]],
                             row_bufs[b].at[pl.ds(k * IDXW, IDXW)], gsems[b])

    def wait_gather(b):
        pltpu.make_async_copy(table_hbm.at[pl.ds(0, CHUNK)], row_bufs[b],
                              gsems[b]).wait()

    def start_out(c, b):
        h = c // cph
        part = c % cph
        row0 = h * BATCH + wid * per_w + part * CHUNK
        pltpu.async_copy(row_bufs[b], s_hbm.at[pl.ds(row0, CHUNK)], osems[b])

    def wait_out(b):
        pltpu.make_async_copy(row_bufs[b], s_hbm.at[pl.ds(0, CHUNK)],
                              osems[b]).wait()

    def step(c, k, do_gather, do_outwait):
        if do_gather:
            bg = (k + 2) % NBUF
            if do_outwait:
                wait_out(bg)
            start_gather(c + 2, bg)
        wait_gather(k)
        start_out(c, k)

    start_gather(0, 0)
    start_gather(1, 1)
    step(0, 0, True, False)
    step(1, 1, True, False)
    step(2, 2, True, True)
    step(3, 3, True, True)

    def outer(i, carry):
        c0 = i * NBUF
        for k in range(NBUF):
            step(c0 + k, k, True, True)
        return carry

    lax.fori_loop(1, n_outer - 1, outer, 0)

    c0 = (n_outer - 1) * NBUF
    step(c0 + 0, 0, True, True)
    step(c0 + 1, 1, True, True)
    step(c0 + 2, 2, False, False)
    step(c0 + 3, 3, False, False)

    for b in range(NBUF):
        wait_out(b)


def _tc_transpose_body(s_ref, o_ref):
    D = o_ref.shape[1]
    half = o_ref.shape[2] // 2
    n, _, w = s_ref.shape
    x2 = s_ref[...].reshape(n * 8, w)
    scale = math.sqrt(D)
    o_ref[0, :, :half] = jnp.transpose(x2[:, :D]) * scale
    o_ref[0, :, half:] = jnp.transpose(x2[:, D:]) * scale


def _tc_transpose_body_alias(s_ref, al_ref, o_ref):
    del al_ref
    _tc_transpose_body(s_ref, o_ref)


def _phase_sizes(hist):
    if hist % 2 or hist < 40:
        return [hist]
    head = max(6, (hist // 10) // 2 * 2)
    mid = hist - 2 * head
    base = (mid // 3) // 2 * 2
    return [head, base, base, mid - 2 * base, head]


def kernel(tokens, table):
    BATCH, HIST = tokens.shape
    V, D = table.shape
    per_w = BATCH // NW
    assert BATCH % (NW * CHUNK) == 0 and D % L == 0
    sizes = _phase_sizes(HIST)
    assert sum(sizes) == HIST
    for s_ in sizes:
        assert (s_ * per_w) % (CHUNK * NBUF) == 0
        assert (s_ * per_w) // CHUNK >= 3 * NBUF

    def make_gather(hp):
        return pl.kernel(
            _sc_gather_body,
            out_type=jax.ShapeDtypeStruct((hp * BATCH, D), jnp.float32),
            mesh=plsc.VectorSubcoreMesh(core_axis_name="c",
                                        subcore_axis_name="s"),
            scratch_types=[
                [pltpu.VMEM((KPC, IDXW), jnp.int32) for _ in range(NBUF)],
                [pltpu.VMEM((CHUNK, D), jnp.float32) for _ in range(NBUF)],
                [pltpu.SemaphoreType.DMA for _ in range(NBUF)],
                [pltpu.SemaphoreType.DMA for _ in range(NBUF)],
            ],
            compiler_params=pltpu.CompilerParams(use_tc_tiling_on_sc=False),
        )

    gathers = {hp: make_gather(hp) for hp in set(sizes)}
    tokens_t = jnp.transpose(tokens)

    rows_per_slab = BATCH * D // (8 * 128)
    out_t = None
    h0 = 0
    for p, hp in enumerate(sizes):
        tok_p = (lax.slice_in_dim(tokens_t, h0, h0 + hp, axis=0)
                 .reshape(hp, 2, BATCH // 2)
                 .transpose(0, 2, 1)
                 .reshape(hp, BATCH // IDXW, IDXW))
        s = gathers[hp](tok_p, table)
        s5 = s.reshape(hp * rows_per_slab, 8, 128)
        in_specs = [pl.BlockSpec((rows_per_slab, 8, 128),
                                 lambda h: (h, 0, 0))]
        operands = [s5]
        kwargs = {}
        body = _tc_transpose_body
        if p > 0:
            in_specs.append(pl.BlockSpec(memory_space=pl.ANY))
            operands.append(out_t)
            kwargs["input_output_aliases"] = {1: 0}
            body = _tc_transpose_body_alias
        out_t = pl.pallas_call(
            body,
            grid=(hp,),
            in_specs=in_specs,
            out_specs=pl.BlockSpec((1, D, BATCH),
                                   lambda h, h0=h0: (h0 + h, 0, 0)),
            out_shape=jax.ShapeDtypeStruct((HIST, D, BATCH), jnp.float32),
            **kwargs,
        )(*operands)
        h0 += hp
    return jnp.transpose(out_t, (2, 0, 1))


# speedup vs baseline: 1.0180x; 1.0072x over previous
import math

import jax
import jax.numpy as jnp
from jax import lax
from jax.experimental import pallas as pl
from jax.experimental.pallas import tpu as pltpu
from jax.experimental.pallas import tpu_sc as plsc

NC, NS, L = 2, 16, 16
NW = NC * NS

IDXW = 128
KPC = 2
CHUNK = KPC * IDXW
NBUF = 4


def _sc_gather_body(tokens_hbm, table_hbm, s_hbm, idx_bufs, row_bufs,
                    gsems, osems):
    D = table_hbm.shape[1]
    HIST, n_idx_rows, _ = tokens_hbm.shape
    BATCH = n_idx_rows * IDXW
    per_w = BATCH // NW
    cph = per_w // CHUNK
    n_chunks = HIST * cph
    n_outer = n_chunks // NBUF
    wid = lax.axis_index("s") * NC + lax.axis_index("c")

    def start_gather(c, b):
        h = c // cph
        part = c % cph
        idx_r0 = wid * (per_w // IDXW) + part * KPC
        pltpu.sync_copy(tokens_hbm.at[h, pl.ds(idx_r0, KPC)], idx_bufs[b])
        for k in range(KPC):
            pltpu.async_copy(table_hbm.at[idx_bufs[b].at[k]],
                             row_bufs[b].at[pl.ds(k * IDXW, IDXW)], gsems[b])

    def wait_gather(b):
        pltpu.make_async_copy(table_hbm.at[pl.ds(0, CHUNK)], row_bufs[b],
                              gsems[b]).wait()

    def start_out(c, b):
        h = c // cph
        part = c % cph
        row0 = h * BATCH + wid * per_w + part * CHUNK
        pltpu.async_copy(row_bufs[b], s_hbm.at[pl.ds(row0, CHUNK)], osems[b])

    def wait_out(b):
        pltpu.make_async_copy(row_bufs[b], s_hbm.at[pl.ds(0, CHUNK)],
                              osems[b]).wait()

    def step(c, k, do_gather, do_outwait):
        if do_gather:
            bg = (k + 2) % NBUF
            if do_outwait:
                wait_out(bg)
            start_gather(c + 2, bg)
        wait_gather(k)
        start_out(c, k)

    start_gather(0, 0)
    start_gather(1, 1)
    step(0, 0, True, False)
    step(1, 1, True, False)
    step(2, 2, True, True)
    step(3, 3, True, True)

    def outer(i, carry):
        c0 = i * NBUF
        for k in range(NBUF):
            step(c0 + k, k, True, True)
        return carry

    lax.fori_loop(1, n_outer - 1, outer, 0)

    c0 = (n_outer - 1) * NBUF
    step(c0 + 0, 0, True, True)
    step(c0 + 1, 1, True, True)
    step(c0 + 2, 2, False, False)
    step(c0 + 3, 3, False, False)

    for b in range(NBUF):
        wait_out(b)


def _tc_transpose_body(s_ref, o_ref):
    D = o_ref.shape[1]
    half = o_ref.shape[2] // 2
    n, _, w = s_ref.shape
    x2 = s_ref[...].reshape(n * 8, w)
    scale = math.sqrt(D)
    o_ref[0, :, :half] = jnp.transpose(x2[:, :D]) * scale
    o_ref[0, :, half:] = jnp.transpose(x2[:, D:]) * scale


def _tc_transpose_body_alias(s_ref, al_ref, o_ref):
    del al_ref
    _tc_transpose_body(s_ref, o_ref)


def _phase_sizes(hist):
    if hist % 8 or hist < 48:
        return [hist]
    return [hist // 4] * 4


def kernel(tokens, table):
    BATCH, HIST = tokens.shape
    V, D = table.shape
    per_w = BATCH // NW
    assert BATCH % (NW * CHUNK) == 0 and D % L == 0
    sizes = _phase_sizes(HIST)
    assert sum(sizes) == HIST
    for s_ in sizes:
        assert (s_ * per_w) % (CHUNK * NBUF) == 0
        assert (s_ * per_w) // CHUNK >= 3 * NBUF

    def make_gather(hp):
        return pl.kernel(
            _sc_gather_body,
            out_type=jax.ShapeDtypeStruct((hp * BATCH, D), jnp.float32),
            mesh=plsc.VectorSubcoreMesh(core_axis_name="c",
                                        subcore_axis_name="s"),
            scratch_types=[
                [pltpu.VMEM((KPC, IDXW), jnp.int32) for _ in range(NBUF)],
                [pltpu.VMEM((CHUNK, D), jnp.float32) for _ in range(NBUF)],
                [pltpu.SemaphoreType.DMA for _ in range(NBUF)],
                [pltpu.SemaphoreType.DMA for _ in range(NBUF)],
            ],
            compiler_params=pltpu.CompilerParams(use_tc_tiling_on_sc=False),
        )

    gathers = {hp: make_gather(hp) for hp in set(sizes)}
    tokens_t = jnp.transpose(tokens)

    rows_per_slab = BATCH * D // (8 * 128)
    out_t = None
    h0 = 0
    for p, hp in enumerate(sizes):
        tok_p = (lax.slice_in_dim(tokens_t, h0, h0 + hp, axis=0)
                 .reshape(hp, 2, BATCH // 2)
                 .transpose(0, 2, 1)
                 .reshape(hp, BATCH // IDXW, IDXW))
        s = gathers[hp](tok_p, table)
        s5 = s.reshape(hp * rows_per_slab, 8, 128)
        in_specs = [pl.BlockSpec((rows_per_slab, 8, 128),
                                 lambda h: (h, 0, 0))]
        operands = [s5]
        kwargs = {}
        body = _tc_transpose_body
        if p > 0:
            in_specs.append(pl.BlockSpec(memory_space=pl.ANY))
            operands.append(out_t)
            kwargs["input_output_aliases"] = {1: 0}
            body = _tc_transpose_body_alias
        out_t = pl.pallas_call(
            body,
            grid=(hp,),
            in_specs=in_specs,
            out_specs=pl.BlockSpec((1, D, BATCH),
                                   lambda h, h0=h0: (h0 + h, 0, 0)),
            out_shape=jax.ShapeDtypeStruct((HIST, D, BATCH), jnp.float32),
            **kwargs,
        )(*operands)
        h0 += hp
    return jnp.transpose(out_t, (2, 0, 1))
